# baseline (speedup 1.0000x reference)
import functools

import jax
import jax.numpy as jnp
from jax import lax
from jax.experimental import pallas as pl
from jax.experimental.pallas import tpu as pltpu
from jax.experimental.pallas import tpu_sc as plsc

L = 16


def _i32(x):
    x &= 0xFFFFFFFF
    return jnp.int32(x - (1 << 32) if x & 0x80000000 else x)


def _make_select(B, S, K):
    NV = S // L
    mesh = plsc.VectorSubcoreMesh(core_axis_name="c", subcore_axis_name="s")
    NC = mesh.num_cores

    @functools.partial(
        pl.kernel,
        out_type=jax.ShapeDtypeStruct((B, K), jnp.int32),
        mesh=mesh,
        compiler_params=pltpu.CompilerParams(needs_layout_passes=False),
        scratch_types=[
            pltpu.VMEM((S,), jnp.int32),
            pltpu.VMEM((256,), jnp.int32),
            pltpu.VMEM((S + L,), jnp.int32),
            pltpu.VMEM((K,), jnp.int32),
            pltpu.VMEM((L,), jnp.int32),
        ],
    )
    def select(keys_hbm, offv_hbm, idx_hbm, keys_v, hist_v, cand_v, idx_v,
               off_v):
        wid = lax.axis_index("s") * NC + lax.axis_index("c")

        @pl.when(wid < B)
        def _():
            b = wid
            pltpu.sync_copy(keys_hbm.at[b], keys_v)
            pltpu.sync_copy(offv_hbm, off_v)
            ov = off_v[...]

            lane = lax.iota(jnp.int32, L)
            ones = jnp.ones((L,), jnp.int32)

            def zero_hist():
                for j in range(16):
                    hist_v[pl.ds(j * L, L)] = jnp.zeros((L,), jnp.int32)

            def hist_scan(krem):
                def scan_body(j, st):
                    carry, dig, krem_new = st
                    c = 15 - j
                    h = hist_v[pl.ds(c * L, L)]
                    srev = jnp.cumsum(jnp.flip(h))
                    s = jnp.flip(srev) + carry
                    tot = jnp.sum(h)
                    s0 = carry + tot
                    in_chunk = jnp.logical_and(carry < krem, s0 >= krem)
                    msk = s >= krem
                    l = jnp.sum(msk.astype(jnp.int32)) - 1
                    sl = jnp.sum(jnp.where(lane == l, s, 0))
                    hl = jnp.sum(jnp.where(lane == l, h, 0))
                    dig = jnp.where(in_chunk, c * L + l, dig)
                    krem_new = jnp.where(in_chunk, krem - (sl - hl), krem_new)
                    return (s0, dig, krem_new)

                _, dig, krem = lax.fori_loop(
                    0, 16, scan_body, (jnp.int32(0), jnp.int32(0), krem)
                )
                return dig, krem

            zero_hist()

            def hist1_body(i, _):
                kv = keys_v[pl.ds(i * L, L)]
                dig = ((kv >> 24) & 0xFF) ^ 0x80
                plsc.addupdate_scatter(hist_v, [dig], ones)
                return 0

            lax.fori_loop(0, NV, hist1_body, 0)
            d1, krem = hist_scan(jnp.int32(K))
            rawb = d1 ^ 0x80
            prefix = rawb << 24

            def cand_body(i, st):
                pos = st
                kv = keys_v[pl.ds(i * L, L)]
                m = ((kv >> 24) & 0xFF) == rawb
                mc = jnp.cumsum(m.astype(jnp.int32))
                plsc.store_scatter(
                    cand_v, [jnp.maximum(pos + mc - 1, 0)], kv, mask=m
                )
                return pos + jnp.sum(m.astype(jnp.int32))

            ncand = lax.fori_loop(0, NV, cand_body, jnp.int32(0))
            pad = jnp.full((L,), 0, jnp.int32) + (prefix ^ _i32(0xFF000000))
            plsc.store_scatter(cand_v, [ncand + lane], pad)
            ncv = (ncand + (L - 1)) // L

            for p in range(1, 4):
                shift = 24 - 8 * p
                prefmask = _i32(0xFFFFFFFF << (shift + 8))
                zero_hist()

                def histp_body(i, _, shift=shift, prefmask=prefmask,
                               prefix=prefix):
                    kv = cand_v[pl.ds(i * L, L)]
                    match = (kv & prefmask) == prefix
                    dig = (kv >> shift) & 0xFF
                    plsc.addupdate_scatter(hist_v, [dig], ones, mask=match)
                    return 0

                lax.fori_loop(0, ncv, histp_body, 0)
                d, krem = hist_scan(krem)
                prefix = prefix | (d << shift)

            thresh = prefix
            need = krem

            lane = lax.iota(jnp.int32, L)

            def comp_body(i, st):
                pos, tt = st
                kv = keys_v[pl.ds(i * L, L)]
                m_gt = kv > thresh
                m_eq = kv == thresh
                eqc = jnp.cumsum(m_eq.astype(jnp.int32))
                keep_eq = jnp.logical_and(m_eq, (tt + eqc) <= need)
                m = jnp.logical_or(m_gt, keep_eq)
                mc = jnp.cumsum(m.astype(jnp.int32))
                pos_v = jnp.clip(pos + mc - 1, 0, K - 1)
                plsc.store_scatter(idx_v, [pos_v], lane + i * L + ov, mask=m)
                return (pos + jnp.sum(m.astype(jnp.int32)),
                        tt + jnp.sum(m_eq.astype(jnp.int32)))

            lax.fori_loop(0, NV, comp_body, (jnp.int32(0), jnp.int32(0)))
            pltpu.sync_copy(idx_v, idx_hbm.at[b])

    return select


def _make_gather(B, S, D, K):
    mesh = plsc.VectorSubcoreMesh(core_axis_name="c", subcore_axis_name="s")
    NC, NS = mesh.num_cores, mesh.num_subcores
    NW = NC * NS
    BK = B * K
    RPT = BK // NW
    CH = 16
    NCH = RPT // CH

    @functools.partial(
        pl.kernel,
        out_type=[
            jax.ShapeDtypeStruct((BK, D), jnp.float32),
            jax.ShapeDtypeStruct((BK,), jnp.int32),
        ],
        mesh=mesh,
        compiler_params=pltpu.CompilerParams(needs_layout_passes=False),
        scratch_types=[
            pltpu.VMEM((RPT,), jnp.int32),
            pltpu.VMEM((NCH, CH), jnp.int32),
            pltpu.VMEM((S,), jnp.int32),
            pltpu.VMEM((RPT,), jnp.int32),
            pltpu.VMEM((CH, D), jnp.float32),
            pltpu.VMEM((CH, D), jnp.float32),
            pltpu.VMEM((CH, D), jnp.float32),
            pltpu.SemaphoreType.DMA,
            pltpu.SemaphoreType.DMA,
            pltpu.SemaphoreType.DMA,
            pltpu.SemaphoreType.DMA,
            pltpu.SemaphoreType.DMA,
            pltpu.SemaphoreType.DMA,
        ],
    )
    def gather(hidden_hbm, mask_hbm, gidx_hbm, out_hbm, pmask_hbm,
               idx_v, idxc_v, mrow_v, pm_v, buf_0, buf_1, buf_2,
               gsem_0, gsem_1, gsem_2, wsem_0, wsem_1, wsem_2):
        wid = lax.axis_index("s") * NC + lax.axis_index("c")
        base = wid * RPT
        b = base // K

        pltpu.sync_copy(gidx_hbm.at[pl.ds(base, RPT)], idx_v)

        boff = b * S

        def idx_body(j, _):
            idxc_v[j] = jnp.clip(idx_v[pl.ds(j * L, L)], 0, S - 1) + boff
            return 0

        lax.fori_loop(0, NCH, idx_body, 0)

        def start_g(c, buf, sem):
            pltpu.async_copy(hidden_hbm.at[idxc_v.at[c]], buf, sem)

        def wait_g(c, buf, sem):
            pltpu.make_async_copy(hidden_hbm.at[idxc_v.at[c]], buf, sem).wait()

        def start_w(c, buf, sem):
            pltpu.async_copy(buf, out_hbm.at[pl.ds(base + c * CH, CH)], sem)

        def wait_w(c, buf, sem):
            pltpu.make_async_copy(
                buf, out_hbm.at[pl.ds(base + c * CH, CH)], sem
            ).wait()

        bufs = (buf_0, buf_1, buf_2)
        gsems = (gsem_0, gsem_1, gsem_2)
        wsems = (wsem_0, wsem_1, wsem_2)

        for k in range(3):
            start_g(k, bufs[k], gsems[k])

        pltpu.sync_copy(mask_hbm.at[b], mrow_v)

        def mg_body(j, _):
            iv = jnp.clip(idx_v[pl.ds(j * L, L)], 0, S - 1)
            pm_v[pl.ds(j * L, L)] = plsc.load_gather(mrow_v, [iv])
            return 0

        lax.fori_loop(0, RPT // L, mg_body, 0)
        pltpu.sync_copy(pm_v, pmask_hbm.at[pl.ds(base, RPT)])

        for k in range(3):
            wait_g(k, bufs[k], gsems[k])
            start_w(k, bufs[k], wsems[k])

        NG = (NCH - 3) // 3

        def group_body(g, _):
            for k in range(3):
                c = 3 + 3 * g + k
                wait_w(c - 3, bufs[k], wsems[k])
                start_g(c, bufs[k], gsems[k])
            for k in range(3):
                c = 3 + 3 * g + k
                wait_g(c, bufs[k], gsems[k])
                start_w(c, bufs[k], wsems[k])
            return 0

        lax.fori_loop(0, NG, group_body, 0)
        done = 3 + 3 * NG
        for t, c in enumerate(range(done, NCH)):
            k = c % 3
            wait_w(c - 3, bufs[k], wsems[k])
            start_g(c, bufs[k], gsems[k])
        for c in range(done, NCH):
            k = c % 3
            wait_g(c, bufs[k], gsems[k])
            start_w(c, bufs[k], wsems[k])
        for c in range(NCH - 3, NCH):
            k = c % 3
            wait_w(c, bufs[k], wsems[k])

    return gather


def kernel(hidden_states, scores, attention_mask, keep_k):
    B, S, D = hidden_states.shape
    K = min(max(1, 4096), S - 1)

    bits = jax.lax.bitcast_convert_type(scores, jnp.int32)
    keys = bits ^ ((bits >> 31) & jnp.int32(0x7FFFFFFF))

    off = jnp.clip(jnp.asarray(keep_k, jnp.int32), 1, S - 1) - jnp.int32(K)
    offv = jnp.full((L,), 1, jnp.int32) * off

    topk_indices = _make_select(B, S, K)(keys, offv)

    hidden_flat = hidden_states.reshape(B * S, D)
    pruned_flat, pmask_flat = _make_gather(B, S, D, K)(
        hidden_flat, attention_mask, topk_indices.reshape(B * K)
    )
    return (
        pruned_flat.reshape(B, K, D),
        pmask_flat.reshape(B, K),
        topk_indices,
    )


# speedup vs baseline: 1.0213x; 1.0213x over previous
import functools

import jax
import jax.numpy as jnp
from jax import lax
from jax.experimental import pallas as pl
from jax.experimental.pallas import tpu as pltpu
from jax.experimental.pallas import tpu_sc as plsc

L = 16


def _i32(x):
    x &= 0xFFFFFFFF
    return jnp.int32(x - (1 << 32) if x & 0x80000000 else x)


def _make_fused(B, S, D, K):
    NV = S // L
    mesh = plsc.VectorSubcoreMesh(core_axis_name="c", subcore_axis_name="s")
    NC, NS = mesh.num_cores, mesh.num_subcores
    NW = NC * NS
    BK = B * K
    BPC = B // NC
    RPT = BK // NW
    CH = 16
    NCH = RPT // CH
    TPB = K // RPT

    @functools.partial(
        pl.kernel,
        out_type=[
            jax.ShapeDtypeStruct((BK, D), jnp.float32),
            jax.ShapeDtypeStruct((BK,), jnp.int32),
            jax.ShapeDtypeStruct((B, K), jnp.int32),
        ],
        mesh=mesh,
        compiler_params=pltpu.CompilerParams(needs_layout_passes=False),
        scratch_types=[
            pltpu.VMEM((S,), jnp.int32),
            pltpu.VMEM((256,), jnp.int32),
            pltpu.VMEM((S + L,), jnp.int32),
            pltpu.VMEM((K,), jnp.int32),
            pltpu.VMEM((L,), jnp.int32),
            pltpu.VMEM((RPT,), jnp.int32),
            pltpu.VMEM((NCH, CH), jnp.int32),
            pltpu.VMEM((S,), jnp.int32),
            pltpu.VMEM((RPT,), jnp.int32),
            pltpu.VMEM((CH, D), jnp.float32),
            pltpu.VMEM((CH, D), jnp.float32),
            pltpu.SemaphoreType.DMA,
            pltpu.SemaphoreType.DMA,
            pltpu.SemaphoreType.DMA,
            pltpu.SemaphoreType.DMA,
        ],
    )
    def fused(keys_hbm, offv_hbm, hidden_hbm, mask_hbm,
              out_hbm, pmask_hbm, idx_hbm,
              keys_v, hist_v, cand_v, idxsel_v, off_v,
              idx_v, idxc_v, mrow_v, pm_v, buf_a, buf_b,
              gsem_a, gsem_b, wsem_a, wsem_b):
        c_ax = lax.axis_index("c")
        s_ax = lax.axis_index("s")

        @pl.when(s_ax < BPC)
        def _():
            b = c_ax * BPC + s_ax
            pltpu.sync_copy(keys_hbm.at[b], keys_v)
            pltpu.sync_copy(offv_hbm, off_v)
            ov = off_v[...]

            lane = lax.iota(jnp.int32, L)
            ones = jnp.ones((L,), jnp.int32)

            def zero_hist():
                for j in range(16):
                    hist_v[pl.ds(j * L, L)] = jnp.zeros((L,), jnp.int32)

            def hist_scan(krem):
                def scan_body(j, st):
                    carry, dig, krem_new = st
                    cc = 15 - j
                    h = hist_v[pl.ds(cc * L, L)]
                    srev = jnp.cumsum(jnp.flip(h))
                    sfx = jnp.flip(srev) + carry
                    tot = jnp.sum(h)
                    s0 = carry + tot
                    in_chunk = jnp.logical_and(carry < krem, s0 >= krem)
                    msk = sfx >= krem
                    l = jnp.sum(msk.astype(jnp.int32)) - 1
                    sl = jnp.sum(jnp.where(lane == l, sfx, 0))
                    hl = jnp.sum(jnp.where(lane == l, h, 0))
                    dig = jnp.where(in_chunk, cc * L + l, dig)
                    krem_new = jnp.where(in_chunk, krem - (sl - hl), krem_new)
                    return (s0, dig, krem_new)

                _, dig, krem = lax.fori_loop(
                    0, 16, scan_body, (jnp.int32(0), jnp.int32(0), krem)
                )
                return dig, krem

            zero_hist()

            def hist1_body(i, _):
                kv = keys_v[pl.ds(i * L, L)]
                dig = ((kv >> 24) & 0xFF) ^ 0x80
                plsc.addupdate_scatter(hist_v, [dig], ones)
                return 0

            lax.fori_loop(0, NV, hist1_body, 0)
            d1, krem = hist_scan(jnp.int32(K))
            rawb = d1 ^ 0x80
            prefix = rawb << 24

            def cand_body(i, pos):
                kv = keys_v[pl.ds(i * L, L)]
                m = ((kv >> 24) & 0xFF) == rawb
                mc = jnp.cumsum(m.astype(jnp.int32))
                plsc.store_scatter(
                    cand_v, [jnp.maximum(pos + mc - 1, 0)], kv, mask=m
                )
                return pos + jnp.sum(m.astype(jnp.int32))

            ncand = lax.fori_loop(0, NV, cand_body, jnp.int32(0))
            pad = jnp.full((L,), 0, jnp.int32) + (prefix ^ _i32(0xFF000000))
            plsc.store_scatter(cand_v, [ncand + lane], pad)
            ncv = (ncand + (L - 1)) // L

            for p in range(1, 4):
                shift = 24 - 8 * p
                prefmask = _i32(0xFFFFFFFF << (shift + 8))
                zero_hist()

                def histp_body(i, _, shift=shift, prefmask=prefmask,
                               prefix=prefix):
                    kv = cand_v[pl.ds(i * L, L)]
                    match = (kv & prefmask) == prefix
                    dig = (kv >> shift) & 0xFF
                    plsc.addupdate_scatter(hist_v, [dig], ones, mask=match)
                    return 0

                lax.fori_loop(0, ncv, histp_body, 0)
                d, krem = hist_scan(krem)
                prefix = prefix | (d << shift)

            thresh = prefix
            need = krem

            def comp_body(i, st):
                pos, tt = st
                kv = keys_v[pl.ds(i * L, L)]
                m_gt = kv > thresh
                m_eq = kv == thresh
                eqc = jnp.cumsum(m_eq.astype(jnp.int32))
                keep_eq = jnp.logical_and(m_eq, (tt + eqc) <= need)
                m = jnp.logical_or(m_gt, keep_eq)
                mc = jnp.cumsum(m.astype(jnp.int32))
                pos_v = jnp.clip(pos + mc - 1, 0, K - 1)
                plsc.store_scatter(idxsel_v, [pos_v], lane + i * L + ov,
                                   mask=m)
                return (pos + jnp.sum(m.astype(jnp.int32)),
                        tt + jnp.sum(m_eq.astype(jnp.int32)))

            lax.fori_loop(0, NV, comp_body, (jnp.int32(0), jnp.int32(0)))
            pltpu.sync_copy(idxsel_v, idx_hbm.at[b])

        plsc.subcore_barrier()

        bt = c_ax * BPC + s_ax // TPB
        col = (s_ax % TPB) * RPT
        base = bt * K + col

        pltpu.sync_copy(idx_hbm.at[bt, pl.ds(col, RPT)], idx_v)

        boff = bt * S

        def idx_body(j, _):
            idxc_v[j] = jnp.clip(idx_v[pl.ds(j * L, L)], 0, S - 1) + boff
            return 0

        lax.fori_loop(0, NCH, idx_body, 0)

        def start_g(c, buf, sem):
            pltpu.async_copy(hidden_hbm.at[idxc_v.at[c]], buf, sem)

        def wait_g(c, buf, sem):
            pltpu.make_async_copy(hidden_hbm.at[idxc_v.at[c]], buf, sem).wait()

        def start_w(c, buf, sem):
            pltpu.async_copy(buf, out_hbm.at[pl.ds(base + c * CH, CH)], sem)

        def wait_w(c, buf, sem):
            pltpu.make_async_copy(
                buf, out_hbm.at[pl.ds(base + c * CH, CH)], sem
            ).wait()

        start_g(0, buf_a, gsem_a)
        start_g(1, buf_b, gsem_b)

        pltpu.sync_copy(mask_hbm.at[bt], mrow_v)

        def mg_body(j, _):
            iv = jnp.clip(idx_v[pl.ds(j * L, L)], 0, S - 1)
            pm_v[pl.ds(j * L, L)] = plsc.load_gather(mrow_v, [iv])
            return 0

        lax.fori_loop(0, RPT // L, mg_body, 0)
        pltpu.sync_copy(pm_v, pmask_hbm.at[pl.ds(base, RPT)])

        wait_g(0, buf_a, gsem_a)
        start_w(0, buf_a, wsem_a)
        wait_g(1, buf_b, gsem_b)
        start_w(1, buf_b, wsem_b)

        def pair_body(g, _):
            c0 = 2 * g
            c1 = c0 + 1
            wait_w(c0 - 2, buf_a, wsem_a)
            start_g(c0, buf_a, gsem_a)
            wait_w(c1 - 2, buf_b, wsem_b)
            start_g(c1, buf_b, gsem_b)
            wait_g(c0, buf_a, gsem_a)
            start_w(c0, buf_a, wsem_a)
            wait_g(c1, buf_b, gsem_b)
            start_w(c1, buf_b, wsem_b)
            return 0

        lax.fori_loop(1, NCH // 2, pair_body, 0)
        wait_w(NCH - 2, buf_a, wsem_a)
        wait_w(NCH - 1, buf_b, wsem_b)

    return fused


def kernel(hidden_states, scores, attention_mask, keep_k):
    B, S, D = hidden_states.shape
    K = min(max(1, 4096), S - 1)

    bits = jax.lax.bitcast_convert_type(scores, jnp.int32)
    keys = bits ^ ((bits >> 31) & jnp.int32(0x7FFFFFFF))

    off = jnp.clip(jnp.asarray(keep_k, jnp.int32), 1, S - 1) - jnp.int32(K)
    offv = jnp.full((L,), 1, jnp.int32) * off

    hidden_flat = hidden_states.reshape(B * S, D)
    pruned_flat, pmask_flat, topk_indices = _make_fused(B, S, D, K)(
        keys, offv, hidden_flat, attention_mask
    )
    return (
        pruned_flat.reshape(B, K, D),
        pmask_flat.reshape(B, K),
        topk_indices,
    )


# speedup vs baseline: 1.0519x; 1.0299x over previous
import functools

import jax
import jax.numpy as jnp
from jax import lax
from jax.experimental import pallas as pl
from jax.experimental.pallas import tpu as pltpu
from jax.experimental.pallas import tpu_sc as plsc

L = 16


def _i32(x):
    x &= 0xFFFFFFFF
    return jnp.int32(x - (1 << 32) if x & 0x80000000 else x)


def _make_fused(B, S, D, K):
    NV = S // L
    mesh = plsc.VectorSubcoreMesh(core_axis_name="c", subcore_axis_name="s")
    NC, NS = mesh.num_cores, mesh.num_subcores
    NW = NC * NS
    BK = B * K
    BPC = B // NC
    RPT = BK // NW
    CH = 16
    NCH = RPT // CH
    TPB = K // RPT
    TPS = NS // BPC
    SLC = S // TPS
    NVS = SLC // L

    @functools.partial(
        pl.kernel,
        out_type=[
            jax.ShapeDtypeStruct((BK, D), jnp.float32),
            jax.ShapeDtypeStruct((BK,), jnp.int32),
            jax.ShapeDtypeStruct((B, K), jnp.int32),
        ],
        mesh=mesh,
        compiler_params=pltpu.CompilerParams(needs_layout_passes=False),
        scratch_types=[
            pltpu.VMEM((S,), jnp.int32),
            pltpu.VMEM((16, 16), jnp.int32),
            pltpu.VMEM((SLC + L,), jnp.int32),
            pltpu.VMEM((K,), jnp.int32),
            pltpu.VMEM((L,), jnp.int32),
            pltpu.VMEM((RPT,), jnp.int32),
            pltpu.VMEM((NCH, CH), jnp.int32),
            pltpu.VMEM((S,), jnp.int32),
            pltpu.VMEM((RPT,), jnp.int32),
            pltpu.VMEM((CH, D), jnp.float32),
            pltpu.VMEM((CH, D), jnp.float32),
            pltpu.VMEM((TPS * 16, 16), jnp.int32),
            pltpu.VMEM_SHARED((4 * BPC * TPS * 16, 16), jnp.int32),
            pltpu.SemaphoreType.DMA,
            pltpu.SemaphoreType.DMA,
            pltpu.SemaphoreType.DMA,
            pltpu.SemaphoreType.DMA,
        ],
    )
    def fused(keys_hbm, offv_hbm, hidden_hbm, mask_hbm,
              out_hbm, pmask_hbm, idx_hbm,
              keys_v, hist_v, cand_v, idxsel_v, off_v,
              idx_v, idxc_v, mrow_v, pm_v, buf_a, buf_b, hsum_v, sh_hist,
              gsem_a, gsem_b, wsem_a, wsem_b):
        c_ax = lax.axis_index("c")
        s_ax = lax.axis_index("s")

        b_loc = s_ax // TPS
        t = s_ax % TPS
        b = c_ax * BPC + b_loc
        is_lead = t == 0

        @pl.when(is_lead)
        def _():
            pltpu.sync_copy(keys_hbm.at[b], keys_v)

        @pl.when(jnp.logical_not(is_lead))
        def _():
            pltpu.sync_copy(
                keys_hbm.at[b, pl.ds(t * SLC, SLC)], keys_v.at[pl.ds(0, SLC)]
            )

        pltpu.sync_copy(offv_hbm, off_v)
        ov = off_v[...]

        lane = lax.iota(jnp.int32, L)
        ones = jnp.ones((L,), jnp.int32)

        def zero_hist():
            for j in range(16):
                hist_v[j] = jnp.zeros((L,), jnp.int32)

        def merge_hist(p):
            wrow = ((p * BPC + b_loc) * TPS + t) * 16
            pltpu.sync_copy(hist_v, sh_hist.at[pl.ds(wrow, 16)])
            plsc.subcore_barrier()
            rrow = (p * BPC + b_loc) * TPS * 16
            pltpu.sync_copy(sh_hist.at[pl.ds(rrow, TPS * 16)], hsum_v)
            for j in range(16):
                acc = hsum_v[j]
                for u in range(1, TPS):
                    acc = acc + hsum_v[u * 16 + j]
                hist_v[j] = acc

        def hist_scan(krem):
            def scan_body(j, st):
                carry, dig, krem_new = st
                cc = 15 - j
                h = hist_v[cc]
                srev = jnp.cumsum(jnp.flip(h))
                sfx = jnp.flip(srev) + carry
                tot = jnp.sum(h)
                s0 = carry + tot
                in_chunk = jnp.logical_and(carry < krem, s0 >= krem)
                msk = sfx >= krem
                l = jnp.sum(msk.astype(jnp.int32)) - 1
                sl = jnp.sum(jnp.where(lane == l, sfx, 0))
                hl = jnp.sum(jnp.where(lane == l, h, 0))
                dig = jnp.where(in_chunk, cc * L + l, dig)
                krem_new = jnp.where(in_chunk, krem - (sl - hl), krem_new)
                return (s0, dig, krem_new)

            _, dig, krem = lax.fori_loop(
                0, 16, scan_body, (jnp.int32(0), jnp.int32(0), krem)
            )
            return dig, krem

        zero_hist()

        def hist1_body(i, _):
            kv = keys_v[pl.ds(i * L, L)]
            dig = ((kv >> 24) & 0xFF) ^ 0x80
            plsc.addupdate_scatter(hist_v, [dig >> 4, dig & 15], ones)
            return 0

        lax.fori_loop(0, NVS, hist1_body, 0)
        merge_hist(0)
        d1, krem = hist_scan(jnp.int32(K))
        rawb = d1 ^ 0x80
        prefix = rawb << 24

        def cand_body(i, pos):
            kv = keys_v[pl.ds(i * L, L)]
            m = ((kv >> 24) & 0xFF) == rawb
            mc = jnp.cumsum(m.astype(jnp.int32))
            plsc.store_scatter(
                cand_v, [jnp.maximum(pos + mc - 1, 0)], kv, mask=m
            )
            return pos + jnp.sum(m.astype(jnp.int32))

        ncand = lax.fori_loop(0, NVS, cand_body, jnp.int32(0))
        pad = jnp.full((L,), 0, jnp.int32) + (prefix ^ _i32(0xFF000000))
        plsc.store_scatter(cand_v, [ncand + lane], pad)
        ncv = (ncand + (L - 1)) // L

        for p in range(1, 4):
            shift = 24 - 8 * p
            prefmask = _i32(0xFFFFFFFF << (shift + 8))
            zero_hist()

            def histp_body(i, _, shift=shift, prefmask=prefmask,
                           prefix=prefix):
                kv = cand_v[pl.ds(i * L, L)]
                match = (kv & prefmask) == prefix
                dig = (kv >> shift) & 0xFF
                plsc.addupdate_scatter(hist_v, [dig >> 4, dig & 15], ones,
                                       mask=match)
                return 0

            lax.fori_loop(0, ncv, histp_body, 0)
            merge_hist(p)
            d, krem = hist_scan(krem)
            prefix = prefix | (d << shift)

        thresh = prefix
        need = krem

        @pl.when(is_lead)
        def _():
            def comp_body(i, st):
                pos, tt = st
                kv = keys_v[pl.ds(i * L, L)]
                m_gt = kv > thresh
                m_eq = kv == thresh
                eqc = jnp.cumsum(m_eq.astype(jnp.int32))
                keep_eq = jnp.logical_and(m_eq, (tt + eqc) <= need)
                m = jnp.logical_or(m_gt, keep_eq)
                mc = jnp.cumsum(m.astype(jnp.int32))
                pos_v = jnp.clip(pos + mc - 1, 0, K - 1)
                plsc.store_scatter(idxsel_v, [pos_v], lane + i * L + ov,
                                   mask=m)
                return (pos + jnp.sum(m.astype(jnp.int32)),
                        tt + jnp.sum(m_eq.astype(jnp.int32)))

            lax.fori_loop(0, NV, comp_body, (jnp.int32(0), jnp.int32(0)))
            pltpu.sync_copy(idxsel_v, idx_hbm.at[b])

        plsc.subcore_barrier()

        bt = c_ax * BPC + s_ax // TPB
        col = (s_ax % TPB) * RPT
        base = bt * K + col

        pltpu.sync_copy(idx_hbm.at[bt, pl.ds(col, RPT)], idx_v)

        boff = bt * S

        def idx_body(j, _):
            idxc_v[j] = jnp.clip(idx_v[pl.ds(j * L, L)], 0, S - 1) + boff
            return 0

        lax.fori_loop(0, NCH, idx_body, 0)

        def start_g(c, buf, sem):
            pltpu.async_copy(hidden_hbm.at[idxc_v.at[c]], buf, sem)

        def wait_g(c, buf, sem):
            pltpu.make_async_copy(hidden_hbm.at[idxc_v.at[c]], buf, sem).wait()

        def start_w(c, buf, sem):
            pltpu.async_copy(buf, out_hbm.at[pl.ds(base + c * CH, CH)], sem)

        def wait_w(c, buf, sem):
            pltpu.make_async_copy(
                buf, out_hbm.at[pl.ds(base + c * CH, CH)], sem
            ).wait()

        start_g(0, buf_a, gsem_a)
        start_g(1, buf_b, gsem_b)

        pltpu.sync_copy(mask_hbm.at[bt], mrow_v)

        def mg_body(j, _):
            iv = jnp.clip(idx_v[pl.ds(j * L, L)], 0, S - 1)
            pm_v[pl.ds(j * L, L)] = plsc.load_gather(mrow_v, [iv])
            return 0

        lax.fori_loop(0, RPT // L, mg_body, 0)
        pltpu.sync_copy(pm_v, pmask_hbm.at[pl.ds(base, RPT)])

        wait_g(0, buf_a, gsem_a)
        start_w(0, buf_a, wsem_a)
        wait_g(1, buf_b, gsem_b)
        start_w(1, buf_b, wsem_b)

        def pair_body(g, _):
            c0 = 2 * g
            c1 = c0 + 1
            wait_w(c0 - 2, buf_a, wsem_a)
            start_g(c0, buf_a, gsem_a)
            wait_w(c1 - 2, buf_b, wsem_b)
            start_g(c1, buf_b, gsem_b)
            wait_g(c0, buf_a, gsem_a)
            start_w(c0, buf_a, wsem_a)
            wait_g(c1, buf_b, gsem_b)
            start_w(c1, buf_b, wsem_b)
            return 0

        lax.fori_loop(1, NCH // 2, pair_body, 0)
        wait_w(NCH - 2, buf_a, wsem_a)
        wait_w(NCH - 1, buf_b, wsem_b)

    return fused


def kernel(hidden_states, scores, attention_mask, keep_k):
    B, S, D = hidden_states.shape
    K = min(max(1, 4096), S - 1)

    bits = jax.lax.bitcast_convert_type(scores, jnp.int32)
    keys = bits ^ ((bits >> 31) & jnp.int32(0x7FFFFFFF))

    off = jnp.clip(jnp.asarray(keep_k, jnp.int32), 1, S - 1) - jnp.int32(K)
    offv = jnp.full((L,), 1, jnp.int32) * off

    hidden_flat = hidden_states.reshape(B * S, D)
    pruned_flat, pmask_flat, topk_indices = _make_fused(B, S, D, K)(
        keys, offv, hidden_flat, attention_mask
    )
    return (
        pruned_flat.reshape(B, K, D),
        pmask_flat.reshape(B, K),
        topk_indices,
    )


# speedup vs baseline: 1.1210x; 1.0657x over previous
import functools

import jax
import jax.numpy as jnp
from jax import lax
from jax.experimental import pallas as pl
from jax.experimental.pallas import tpu as pltpu
from jax.experimental.pallas import tpu_sc as plsc

L = 16


def _i32(x):
    x &= 0xFFFFFFFF
    return jnp.int32(x - (1 << 32) if x & 0x80000000 else x)


def _make_fused(B, S, D, K):
    NV = S // L
    mesh = plsc.VectorSubcoreMesh(core_axis_name="c", subcore_axis_name="s")
    NC, NS = mesh.num_cores, mesh.num_subcores
    NW = NC * NS
    BK = B * K
    BPC = B // NC
    RPT = BK // NW
    CH = 16
    NCH = RPT // CH
    TPB = K // RPT
    TPS = NS // BPC
    SLC = S // TPS
    NVS = SLC // L

    @functools.partial(
        pl.kernel,
        out_type=[
            jax.ShapeDtypeStruct((BK, D), jnp.float32),
            jax.ShapeDtypeStruct((BK,), jnp.int32),
            jax.ShapeDtypeStruct((B, K), jnp.int32),
        ],
        mesh=mesh,
        compiler_params=pltpu.CompilerParams(needs_layout_passes=False),
        scratch_types=[
            pltpu.VMEM((SLC,), jnp.int32),
            pltpu.VMEM((16, 16), jnp.int32),
            pltpu.VMEM((SLC + L,), jnp.int32),
            pltpu.VMEM((SLC + L,), jnp.int32),
            pltpu.VMEM((L,), jnp.int32),
            pltpu.VMEM((L,), jnp.int32),
            pltpu.VMEM((TPS, 16), jnp.int32),
            pltpu.VMEM((TPS * SLC,), jnp.int32),
            pltpu.VMEM((RPT,), jnp.int32),
            pltpu.VMEM((NCH, CH), jnp.int32),
            pltpu.VMEM((S,), jnp.int32),
            pltpu.VMEM((RPT,), jnp.int32),
            pltpu.VMEM((CH, D), jnp.float32),
            pltpu.VMEM((CH, D), jnp.float32),
            pltpu.VMEM((TPS * 16, 16), jnp.int32),
            pltpu.VMEM_SHARED((4 * BPC * TPS * 16, 16), jnp.int32),
            pltpu.VMEM_SHARED((BPC * TPS, 16), jnp.int32),
            pltpu.VMEM_SHARED((BPC * TPS * SLC,), jnp.int32),
            pltpu.SemaphoreType.DMA,
            pltpu.SemaphoreType.DMA,
            pltpu.SemaphoreType.DMA,
            pltpu.SemaphoreType.DMA,
        ],
    )
    def fused(keys_hbm, offv_hbm, hidden_hbm, mask_hbm,
              out_hbm, pmask_hbm, idx_hbm,
              keys_v, hist_v, cand_v, idxloc_v, off_v, cnt_v, cnts_v, seg_v,
              idx_v, idxc_v, mrow_v, pm_v, buf_a, buf_b, hsum_v,
              sh_hist, sh_cnt, sh_idx,
              gsem_a, gsem_b, wsem_a, wsem_b):
        c_ax = lax.axis_index("c")
        s_ax = lax.axis_index("s")

        b_loc = s_ax // TPS
        t = s_ax % TPS
        b = c_ax * BPC + b_loc

        pltpu.sync_copy(keys_hbm.at[b, pl.ds(t * SLC, SLC)], keys_v)
        pltpu.sync_copy(offv_hbm, off_v)
        ov = off_v[...]

        lane = lax.iota(jnp.int32, L)
        ones = jnp.ones((L,), jnp.int32)

        def zero_hist():
            for j in range(16):
                hist_v[j] = jnp.zeros((L,), jnp.int32)

        def merge_hist(p):
            wrow = ((p * BPC + b_loc) * TPS + t) * 16
            pltpu.sync_copy(hist_v, sh_hist.at[pl.ds(wrow, 16)])
            plsc.subcore_barrier()
            rrow = (p * BPC + b_loc) * TPS * 16
            pltpu.sync_copy(sh_hist.at[pl.ds(rrow, TPS * 16)], hsum_v)
            for j in range(16):
                acc = hsum_v[j]
                for u in range(1, TPS):
                    acc = acc + hsum_v[u * 16 + j]
                hist_v[j] = acc

        def hist_scan(krem):
            def scan_body(j, st):
                carry, dig, krem_new = st
                cc = 15 - j
                h = hist_v[cc]
                srev = jnp.cumsum(jnp.flip(h))
                sfx = jnp.flip(srev) + carry
                tot = jnp.sum(h)
                s0 = carry + tot
                in_chunk = jnp.logical_and(carry < krem, s0 >= krem)
                msk = sfx >= krem
                l = jnp.sum(msk.astype(jnp.int32)) - 1
                sl = jnp.sum(jnp.where(lane == l, sfx, 0))
                hl = jnp.sum(jnp.where(lane == l, h, 0))
                dig = jnp.where(in_chunk, cc * L + l, dig)
                krem_new = jnp.where(in_chunk, krem - (sl - hl), krem_new)
                return (s0, dig, krem_new)

            _, dig, krem = lax.fori_loop(
                0, 16, scan_body, (jnp.int32(0), jnp.int32(0), krem)
            )
            return dig, krem

        zero_hist()

        def hist1_body(i, _):
            kv = keys_v[pl.ds(i * L, L)]
            dig = ((kv >> 24) & 0xFF) ^ 0x80
            plsc.addupdate_scatter(hist_v, [dig >> 4, dig & 15], ones)
            return 0

        lax.fori_loop(0, NVS, hist1_body, 0)
        merge_hist(0)
        d1, krem = hist_scan(jnp.int32(K))
        rawb = d1 ^ 0x80
        prefix = rawb << 24

        def cand_body(i, pos):
            kv = keys_v[pl.ds(i * L, L)]
            m = ((kv >> 24) & 0xFF) == rawb
            mc = jnp.cumsum(m.astype(jnp.int32))
            plsc.store_scatter(
                cand_v, [jnp.maximum(pos + mc - 1, 0)], kv, mask=m
            )
            return pos + jnp.sum(m.astype(jnp.int32))

        ncand = lax.fori_loop(0, NVS, cand_body, jnp.int32(0))
        pad = jnp.full((L,), 0, jnp.int32) + (prefix ^ _i32(0xFF000000))
        plsc.store_scatter(cand_v, [ncand + lane], pad)
        ncv = (ncand + (L - 1)) // L

        for p in range(1, 4):
            shift = 24 - 8 * p
            prefmask = _i32(0xFFFFFFFF << (shift + 8))
            zero_hist()

            def histp_body(i, _, shift=shift, prefmask=prefmask,
                           prefix=prefix):
                kv = cand_v[pl.ds(i * L, L)]
                match = (kv & prefmask) == prefix
                dig = (kv >> shift) & 0xFF
                plsc.addupdate_scatter(hist_v, [dig >> 4, dig & 15], ones,
                                       mask=match)
                return 0

            lax.fori_loop(0, ncv, histp_body, 0)
            merge_hist(p)
            d, krem = hist_scan(krem)
            prefix = prefix | (d << shift)

        thresh = prefix
        need = krem

        def cnt_body(i, st):
            g, e = st
            kv = keys_v[pl.ds(i * L, L)]
            g = g + jnp.sum((kv > thresh).astype(jnp.int32))
            e = e + jnp.sum((kv == thresh).astype(jnp.int32))
            return (g, e)

        ngt, neq = lax.fori_loop(0, NVS, cnt_body,
                                 (jnp.int32(0), jnp.int32(0)))

        cnt_v[...] = (jnp.where(lane == 0, ngt, 0)
                      + jnp.where(lane == 1, neq, 0))
        pltpu.sync_copy(cnt_v, sh_cnt.at[b_loc * TPS + t])
        plsc.subcore_barrier()
        pltpu.sync_copy(sh_cnt.at[pl.ds(b_loc * TPS, TPS)], cnts_v)

        bases = []
        ks = []
        acc_e = jnp.int32(0)
        acc_k = jnp.int32(0)
        my_need = jnp.int32(0)
        for u in range(TPS):
            row = cnts_v[u]
            gt_u = jnp.sum(jnp.where(lane == 0, row, 0))
            eq_u = jnp.sum(jnp.where(lane == 1, row, 0))
            need_u = jnp.clip(need - acc_e, 0, eq_u)
            k_u = gt_u + need_u
            bases.append(acc_k)
            ks.append(k_u)
            my_need = jnp.where(t == u, need_u, my_need)
            acc_e = acc_e + eq_u
            acc_k = acc_k + k_u

        gofs = t * SLC

        def comp_body(i, st):
            pos, tt = st
            kv = keys_v[pl.ds(i * L, L)]
            m_gt = kv > thresh
            m_eq = kv == thresh
            eqc = jnp.cumsum(m_eq.astype(jnp.int32))
            keep_eq = jnp.logical_and(m_eq, (tt + eqc) <= my_need)
            m = jnp.logical_or(m_gt, keep_eq)
            mc = jnp.cumsum(m.astype(jnp.int32))
            pos_v = jnp.clip(pos + mc - 1, 0, SLC + L - 1)
            plsc.store_scatter(idxloc_v, [pos_v], lane + i * L + gofs + ov,
                               mask=m)
            return (pos + jnp.sum(m.astype(jnp.int32)),
                    tt + jnp.sum(m_eq.astype(jnp.int32)))

        lax.fori_loop(0, NVS, comp_body, (jnp.int32(0), jnp.int32(0)))

        pltpu.sync_copy(idxloc_v.at[pl.ds(0, SLC)],
                        sh_idx.at[pl.ds((b_loc * TPS + t) * SLC, SLC)])
        plsc.subcore_barrier()
        pltpu.sync_copy(sh_idx.at[pl.ds(b_loc * TPS * SLC, TPS * SLC)],
                        seg_v)

        col = t * RPT
        base = b * K + col

        def asm_body(i, _):
            j = col + i * L + lane
            flat = j
            for u in range(1, TPS):
                flat = flat + (j >= bases[u]).astype(jnp.int32) * (
                    SLC - ks[u - 1]
                )
            idx_v[pl.ds(i * L, L)] = plsc.load_gather(seg_v, [flat])
            return 0

        lax.fori_loop(0, RPT // L, asm_body, 0)
        pltpu.sync_copy(idx_v, idx_hbm.at[b, pl.ds(col, RPT)])

        boff = b * S

        def idx_body(j, _):
            idxc_v[j] = jnp.clip(idx_v[pl.ds(j * L, L)], 0, S - 1) + boff
            return 0

        lax.fori_loop(0, NCH, idx_body, 0)

        def start_g(c, buf, sem):
            pltpu.async_copy(hidden_hbm.at[idxc_v.at[c]], buf, sem)

        def wait_g(c, buf, sem):
            pltpu.make_async_copy(hidden_hbm.at[idxc_v.at[c]], buf, sem).wait()

        def start_w(c, buf, sem):
            pltpu.async_copy(buf, out_hbm.at[pl.ds(base + c * CH, CH)], sem)

        def wait_w(c, buf, sem):
            pltpu.make_async_copy(
                buf, out_hbm.at[pl.ds(base + c * CH, CH)], sem
            ).wait()

        start_g(0, buf_a, gsem_a)
        start_g(1, buf_b, gsem_b)

        pltpu.sync_copy(mask_hbm.at[b], mrow_v)

        def mg_body(j, _):
            iv = jnp.clip(idx_v[pl.ds(j * L, L)], 0, S - 1)
            pm_v[pl.ds(j * L, L)] = plsc.load_gather(mrow_v, [iv])
            return 0

        lax.fori_loop(0, RPT // L, mg_body, 0)
        pltpu.sync_copy(pm_v, pmask_hbm.at[pl.ds(base, RPT)])

        wait_g(0, buf_a, gsem_a)
        start_w(0, buf_a, wsem_a)
        wait_g(1, buf_b, gsem_b)
        start_w(1, buf_b, wsem_b)

        def pair_body(g, _):
            c0 = 2 * g
            c1 = c0 + 1
            wait_w(c0 - 2, buf_a, wsem_a)
            start_g(c0, buf_a, gsem_a)
            wait_w(c1 - 2, buf_b, wsem_b)
            start_g(c1, buf_b, gsem_b)
            wait_g(c0, buf_a, gsem_a)
            start_w(c0, buf_a, wsem_a)
            wait_g(c1, buf_b, gsem_b)
            start_w(c1, buf_b, wsem_b)
            return 0

        lax.fori_loop(1, NCH // 2, pair_body, 0)
        wait_w(NCH - 2, buf_a, wsem_a)
        wait_w(NCH - 1, buf_b, wsem_b)

    return fused


def kernel(hidden_states, scores, attention_mask, keep_k):
    B, S, D = hidden_states.shape
    K = min(max(1, 4096), S - 1)

    bits = jax.lax.bitcast_convert_type(scores, jnp.int32)
    keys = bits ^ ((bits >> 31) & jnp.int32(0x7FFFFFFF))

    off = jnp.clip(jnp.asarray(keep_k, jnp.int32), 1, S - 1) - jnp.int32(K)
    offv = jnp.full((L,), 1, jnp.int32) * off

    hidden_flat = hidden_states.reshape(B * S, D)
    pruned_flat, pmask_flat, topk_indices = _make_fused(B, S, D, K)(
        keys, offv, hidden_flat, attention_mask
    )
    return (
        pruned_flat.reshape(B, K, D),
        pmask_flat.reshape(B, K),
        topk_indices,
    )
